# baseline (speedup 1.0000x reference)
import functools

import jax
import jax.numpy as jnp
from jax import lax
from jax.experimental import pallas as pl
from jax.experimental.pallas import tpu as pltpu
from jax.experimental.pallas import tpu_sc as plsc

OUT_H = 7
OUT_W = 7
SPATIAL_SCALE = 0.25
SAMPLING_RATIO = 2
LANES = 16


@functools.lru_cache(maxsize=None)
def _make_sc_call(R, C, NROWS):
    info = plsc.get_sparse_core_info()
    NC, NS = info.num_cores, info.num_subcores
    NW = NC * NS
    RPW = R // NW
    NBINS = OUT_H * OUT_W
    BPC = 7
    NCHUNK = NBINS // BPC
    G = SAMPLING_RATIO * SAMPLING_RATIO * 4
    ROWS = BPC * G
    CH = C // LANES

    HC = C // 2
    CG = HC // LANES

    mesh = plsc.VectorSubcoreMesh(core_axis_name="c", subcore_axis_name="s")

    @functools.partial(
        pl.kernel,
        out_type=jax.ShapeDtypeStruct((R, C, NBINS), jnp.float32),
        mesh=mesh,
        compiler_params=pltpu.CompilerParams(needs_layout_passes=False),
        scratch_types=[
            pltpu.VMEM((NCHUNK, ROWS), jnp.int32),
            pltpu.VMEM((NCHUNK, ROWS), jnp.float32),
            pltpu.VMEM((ROWS, HC), jnp.int32),
            pltpu.VMEM((ROWS, HC), jnp.int32),
            pltpu.VMEM((C, NBINS), jnp.float32),
            pltpu.SemaphoreType.DMA,
            pltpu.SemaphoreType.DMA,
        ],
    )
    def roi_align_sc(feat_hbm, idx_hbm, w_hbm, out_hbm,
                     idx_v, w_v, buf0, buf1, out_t_v, sem_g0, sem_g1):
        wid = lax.axis_index("s") * NC + lax.axis_index("c")
        bufs = (buf0, buf1)
        sems = (sem_g0, sem_g1)
        mask_hi = jnp.int32(-65536)
        lane = lax.iota(jnp.int32, LANES)

        def roi_body(t, carry):
            r = wid * RPW + t
            pltpu.sync_copy(idx_hbm.at[r], idx_v)
            pltpu.sync_copy(w_hbm.at[r], w_v)
            pltpu.async_copy(feat_hbm.at[idx_v.at[0]], bufs[0], sems[0])
            for c in range(NCHUNK):
                if c + 1 < NCHUNK:
                    pltpu.async_copy(feat_hbm.at[idx_v.at[c + 1]],
                                     bufs[(c + 1) % 2], sems[(c + 1) % 2])
                pltpu.make_async_copy(feat_hbm.at[idx_v.at[c]],
                                      bufs[c % 2], sems[c % 2]).wait()
                buf_v = bufs[c % 2]

                def bin_body(bloc, _, c=c, buf_v=buf_v):
                    g = c * BPC + bloc
                    rowbase = bloc * G
                    wvec = w_v[c, pl.ds(rowbase, G)]
                    acc_lo = [None] * CG
                    acc_hi = [None] * CG
                    for k in range(G):
                        w = wvec[k]
                        for gc in range(CG):
                            v = buf_v[rowbase + k, pl.ds(gc * LANES, LANES)]
                            lo = plsc.bitcast(v << 16, jnp.float32)
                            hi = plsc.bitcast(v & mask_hi, jnp.float32)
                            if k == 0:
                                acc_lo[gc] = w * lo
                                acc_hi[gc] = w * hi
                            else:
                                acc_lo[gc] = acc_lo[gc] + w * lo
                                acc_hi[gc] = acc_hi[gc] + w * hi
                    vg = jnp.full((LANES,), g, jnp.int32)
                    for gc in range(CG):
                        tgt = gc * LANES + lane
                        plsc.store_scatter(out_t_v, [tgt, vg], acc_lo[gc])
                        plsc.store_scatter(out_t_v, [tgt + HC, vg],
                                           acc_hi[gc])
                    return 0

                lax.fori_loop(0, BPC, bin_body, 0)
            pltpu.sync_copy(out_t_v, out_hbm.at[r])
            return 0

        lax.fori_loop(0, RPW, roi_body, 0)

    return roi_align_sc


def kernel(features, rois):
    N, C, H, W = features.shape
    R = rois.shape[0]
    u = jax.lax.bitcast_convert_type(features, jnp.uint32)
    r = (u + 0x7FFF + ((u >> 16) & 1)) >> 16
    word = jax.lax.bitcast_convert_type(
        r[:, :C // 2] | (r[:, C // 2:] << 16), jnp.int32)
    feat = jnp.transpose(word, (0, 2, 3, 1)).reshape(N * H * W, C // 2)

    b = rois[:, 0].astype(jnp.int32)
    start_w = rois[:, 1] * SPATIAL_SCALE - 0.5
    start_h = rois[:, 2] * SPATIAL_SCALE - 0.5
    roi_w = rois[:, 3] * SPATIAL_SCALE - 0.5 - start_w
    roi_h = rois[:, 4] * SPATIAL_SCALE - 0.5 - start_h
    bin_h = roi_h / OUT_H
    bin_w = roi_w / OUT_W
    gh = gw = SAMPLING_RATIO
    ph = jnp.arange(OUT_H, dtype=jnp.float32)
    pw = jnp.arange(OUT_W, dtype=jnp.float32)
    iy = jnp.arange(gh, dtype=jnp.float32)
    ix = jnp.arange(gw, dtype=jnp.float32)
    y = (start_h[:, None, None] + ph[None, :, None] * bin_h[:, None, None]
         + (iy[None, None, :] + 0.5) * bin_h[:, None, None] / gh)
    x = (start_w[:, None, None] + pw[None, :, None] * bin_w[:, None, None]
         + (ix[None, None, :] + 0.5) * bin_w[:, None, None] / gw)
    P = OUT_H * OUT_W * gh * gw
    yy = jnp.broadcast_to(y[:, :, None, :, None],
                          (R, OUT_H, OUT_W, gh, gw)).reshape(R, P)
    xx = jnp.broadcast_to(x[:, None, :, None, :],
                          (R, OUT_H, OUT_W, gh, gw)).reshape(R, P)
    valid = (yy > -1.0) & (yy < H) & (xx > -1.0) & (xx < W)
    yc = jnp.clip(yy, 0.0, H - 1)
    xc = jnp.clip(xx, 0.0, W - 1)
    y_low = jnp.floor(yc).astype(jnp.int32)
    x_low = jnp.floor(xc).astype(jnp.int32)
    y_high = jnp.minimum(y_low + 1, H - 1)
    x_high = jnp.minimum(x_low + 1, W - 1)
    ly = yc - y_low.astype(jnp.float32)
    lx = xc - x_low.astype(jnp.float32)
    hy = 1.0 - ly
    hx = 1.0 - lx
    base = (b * (H * W))[:, None]
    idx4 = jnp.stack(
        [base + y_low * W + x_low, base + y_low * W + x_high,
         base + y_high * W + x_low, base + y_high * W + x_high], axis=-1)
    scale = jnp.where(valid, 1.0 / (gh * gw), 0.0)
    w4 = jnp.stack([hy * hx, hy * lx, ly * hx, ly * lx], axis=-1) * scale[:, :, None]

    NROWS = P * 4
    NCHUNK = 7
    idx_all = idx4.reshape(R, NCHUNK, NROWS // NCHUNK).astype(jnp.int32)
    w_all = w4.reshape(R, NCHUNK, NROWS // NCHUNK).astype(jnp.float32)

    sc_call = _make_sc_call(R, C, NROWS)
    out = sc_call(feat, idx_all, w_all)
    return out.reshape(R, C, OUT_H, OUT_W)


# speedup vs baseline: 1.2062x; 1.2062x over previous
import functools

import jax
import jax.numpy as jnp
from jax import lax
from jax.experimental import pallas as pl
from jax.experimental.pallas import tpu as pltpu
from jax.experimental.pallas import tpu_sc as plsc

OUT_H = 7
OUT_W = 7
SPATIAL_SCALE = 0.25
SAMPLING_RATIO = 2
LANES = 16


@functools.lru_cache(maxsize=None)
def _make_sc_call(R, C, NROWS):
    info = plsc.get_sparse_core_info()
    NC, NS = info.num_cores, info.num_subcores
    NW = NC * NS
    RPW = R // NW
    NBINS = OUT_H * OUT_W
    BPC = 7
    NCHUNK = NBINS // BPC
    G = SAMPLING_RATIO * SAMPLING_RATIO * 4
    ROWS = BPC * G
    CH = C // LANES

    HC = C // 2
    CG = HC // LANES

    mesh = plsc.VectorSubcoreMesh(core_axis_name="c", subcore_axis_name="s")

    @functools.partial(
        pl.kernel,
        out_type=jax.ShapeDtypeStruct((R, C * NBINS), jnp.float32),
        mesh=mesh,
        compiler_params=pltpu.CompilerParams(needs_layout_passes=False),
        scratch_types=[
            pltpu.VMEM((NCHUNK, ROWS), jnp.int32),
            pltpu.VMEM((NCHUNK, ROWS), jnp.float32),
            pltpu.VMEM((ROWS, HC), jnp.int32),
            pltpu.VMEM((ROWS, HC), jnp.int32),
            pltpu.VMEM((C * NBINS,), jnp.float32),
            pltpu.SemaphoreType.DMA,
            pltpu.SemaphoreType.DMA,
        ],
    )
    def roi_align_sc(feat_hbm, idx_hbm, w_hbm, out_hbm,
                     idx_v, w_v, buf0, buf1, out_t_v, sem_g0, sem_g1):
        wid = lax.axis_index("s") * NC + lax.axis_index("c")
        bufs = (buf0, buf1)
        sems = (sem_g0, sem_g1)
        mask_hi = jnp.int32(-65536)
        lane = lax.iota(jnp.int32, LANES)

        def roi_body(t, carry):
            r = wid * RPW + t
            pltpu.sync_copy(idx_hbm.at[r], idx_v)
            pltpu.sync_copy(w_hbm.at[r], w_v)
            pltpu.async_copy(feat_hbm.at[idx_v.at[0]], bufs[0], sems[0])
            for c in range(NCHUNK):
                if c + 1 < NCHUNK:
                    pltpu.async_copy(feat_hbm.at[idx_v.at[c + 1]],
                                     bufs[(c + 1) % 2], sems[(c + 1) % 2])
                pltpu.make_async_copy(feat_hbm.at[idx_v.at[c]],
                                      bufs[c % 2], sems[c % 2]).wait()
                buf_v = bufs[c % 2]

                def bin_body(bloc, _, c=c, buf_v=buf_v):
                    g = c * BPC + bloc
                    rowbase = bloc * G
                    wvec = w_v[c, pl.ds(rowbase, G)]
                    acc_lo = [None] * CG
                    acc_hi = [None] * CG
                    for k in range(G):
                        w = wvec[k]
                        for gc in range(CG):
                            v = buf_v[rowbase + k, pl.ds(gc * LANES, LANES)]
                            lo = plsc.bitcast(v << 16, jnp.float32)
                            hi = plsc.bitcast(v & mask_hi, jnp.float32)
                            if k == 0:
                                acc_lo[gc] = w * lo
                                acc_hi[gc] = w * hi
                            else:
                                acc_lo[gc] = acc_lo[gc] + w * lo
                                acc_hi[gc] = acc_hi[gc] + w * hi
                    for gc in range(CG):
                        tgt = (gc * LANES + lane) * NBINS + g
                        plsc.store_scatter(out_t_v, [tgt], acc_lo[gc])
                        plsc.store_scatter(out_t_v, [tgt + HC * NBINS],
                                           acc_hi[gc])
                    return 0

                lax.fori_loop(0, BPC, bin_body, 0)
            pltpu.sync_copy(out_t_v, out_hbm.at[r])
            return 0

        lax.fori_loop(0, RPW, roi_body, 0)

    return roi_align_sc


def kernel(features, rois):
    N, C, H, W = features.shape
    R = rois.shape[0]
    u = jax.lax.bitcast_convert_type(features, jnp.uint32)
    r = (u + 0x7FFF + ((u >> 16) & 1)) >> 16
    word = jax.lax.bitcast_convert_type(
        r[:, :C // 2] | (r[:, C // 2:] << 16), jnp.int32)
    feat = jnp.transpose(word, (0, 2, 3, 1)).reshape(N * H * W, C // 2)

    b = rois[:, 0].astype(jnp.int32)
    start_w = rois[:, 1] * SPATIAL_SCALE - 0.5
    start_h = rois[:, 2] * SPATIAL_SCALE - 0.5
    roi_w = rois[:, 3] * SPATIAL_SCALE - 0.5 - start_w
    roi_h = rois[:, 4] * SPATIAL_SCALE - 0.5 - start_h
    bin_h = roi_h / OUT_H
    bin_w = roi_w / OUT_W
    gh = gw = SAMPLING_RATIO
    ph = jnp.arange(OUT_H, dtype=jnp.float32)
    pw = jnp.arange(OUT_W, dtype=jnp.float32)
    iy = jnp.arange(gh, dtype=jnp.float32)
    ix = jnp.arange(gw, dtype=jnp.float32)
    y = (start_h[:, None, None] + ph[None, :, None] * bin_h[:, None, None]
         + (iy[None, None, :] + 0.5) * bin_h[:, None, None] / gh)
    x = (start_w[:, None, None] + pw[None, :, None] * bin_w[:, None, None]
         + (ix[None, None, :] + 0.5) * bin_w[:, None, None] / gw)
    P = OUT_H * OUT_W * gh * gw
    yy = jnp.broadcast_to(y[:, :, None, :, None],
                          (R, OUT_H, OUT_W, gh, gw)).reshape(R, P)
    xx = jnp.broadcast_to(x[:, None, :, None, :],
                          (R, OUT_H, OUT_W, gh, gw)).reshape(R, P)
    valid = (yy > -1.0) & (yy < H) & (xx > -1.0) & (xx < W)
    yc = jnp.clip(yy, 0.0, H - 1)
    xc = jnp.clip(xx, 0.0, W - 1)
    y_low = jnp.floor(yc).astype(jnp.int32)
    x_low = jnp.floor(xc).astype(jnp.int32)
    y_high = jnp.minimum(y_low + 1, H - 1)
    x_high = jnp.minimum(x_low + 1, W - 1)
    ly = yc - y_low.astype(jnp.float32)
    lx = xc - x_low.astype(jnp.float32)
    hy = 1.0 - ly
    hx = 1.0 - lx
    base = (b * (H * W))[:, None]
    idx4 = jnp.stack(
        [base + y_low * W + x_low, base + y_low * W + x_high,
         base + y_high * W + x_low, base + y_high * W + x_high], axis=-1)
    scale = jnp.where(valid, 1.0 / (gh * gw), 0.0)
    w4 = jnp.stack([hy * hx, hy * lx, ly * hx, ly * lx], axis=-1) * scale[:, :, None]

    NROWS = P * 4
    NCHUNK = 7
    idx_all = idx4.reshape(R, NCHUNK, NROWS // NCHUNK).astype(jnp.int32)
    w_all = w4.reshape(R, NCHUNK, NROWS // NCHUNK).astype(jnp.float32)

    sc_call = _make_sc_call(R, C, NROWS)
    out = sc_call(feat, idx_all, w_all)
    return out.reshape(R, C, OUT_H, OUT_W)


# speedup vs baseline: 1.2754x; 1.0574x over previous
import functools

import jax
import jax.numpy as jnp
from jax import lax
from jax.experimental import pallas as pl
from jax.experimental.pallas import tpu as pltpu
from jax.experimental.pallas import tpu_sc as plsc

OUT_H = 7
OUT_W = 7
SPATIAL_SCALE = 0.25
SAMPLING_RATIO = 2
LANES = 16


@functools.lru_cache(maxsize=None)
def _make_sc_call(R, C, NROWS):
    info = plsc.get_sparse_core_info()
    NC, NS = info.num_cores, info.num_subcores
    NW = NC * NS
    RPW = R // NW
    NBINS = OUT_H * OUT_W
    BPC = 7
    NCHUNK = NBINS // BPC
    G = SAMPLING_RATIO * SAMPLING_RATIO * 4
    ROWS = BPC * G
    CH = C // LANES

    HC = C // 2
    CG = HC // LANES

    mesh = plsc.VectorSubcoreMesh(core_axis_name="c", subcore_axis_name="s")

    @functools.partial(
        pl.kernel,
        out_type=jax.ShapeDtypeStruct((R, C * NBINS), jnp.float32),
        mesh=mesh,
        compiler_params=pltpu.CompilerParams(needs_layout_passes=False),
        scratch_types=[
            pltpu.VMEM((NCHUNK, ROWS), jnp.int32),
            pltpu.VMEM((NCHUNK, ROWS), jnp.float32),
            pltpu.VMEM((ROWS, HC), jnp.int32),
            pltpu.VMEM((ROWS, HC), jnp.int32),
            pltpu.VMEM((C * NBINS,), jnp.float32),
            pltpu.SemaphoreType.DMA,
            pltpu.SemaphoreType.DMA,
        ],
    )
    def roi_align_sc(feat_hbm, idx_hbm, w_hbm, out_hbm,
                     idx_v, w_v, buf0, buf1, out_t_v, sem_g0, sem_g1):
        wid = lax.axis_index("s") * NC + lax.axis_index("c")
        bufs = (buf0, buf1)
        sems = (sem_g0, sem_g1)
        mask_hi = jnp.int32(-65536)
        lane = lax.iota(jnp.int32, LANES)

        def roi_body(t, carry):
            r = wid * RPW + t
            pltpu.sync_copy(idx_hbm.at[r], idx_v)
            pltpu.sync_copy(w_hbm.at[r], w_v)
            pltpu.async_copy(feat_hbm.at[idx_v.at[0]], bufs[0], sems[0])
            for c in range(NCHUNK):
                if c + 1 < NCHUNK:
                    pltpu.async_copy(feat_hbm.at[idx_v.at[c + 1]],
                                     bufs[(c + 1) % 2], sems[(c + 1) % 2])
                pltpu.make_async_copy(feat_hbm.at[idx_v.at[c]],
                                      bufs[c % 2], sems[c % 2]).wait()
                buf_v = bufs[c % 2]

                def bin_body(bloc, _, c=c, buf_v=buf_v):
                    g = c * BPC + bloc
                    rowbase = bloc * G
                    wvec = w_v[c, pl.ds(rowbase, G)]
                    acc_lo = [None] * CG
                    acc_hi = [None] * CG
                    for k in range(G):
                        w = wvec[k]
                        for gc in range(CG):
                            v = buf_v[rowbase + k, pl.ds(gc * LANES, LANES)]
                            lo = plsc.bitcast(v << 16, jnp.float32)
                            hi = plsc.bitcast(v & mask_hi, jnp.float32)
                            if k == 0:
                                acc_lo[gc] = w * lo
                                acc_hi[gc] = w * hi
                            else:
                                acc_lo[gc] = acc_lo[gc] + w * lo
                                acc_hi[gc] = acc_hi[gc] + w * hi
                    for gc in range(CG):
                        tgt = (gc * LANES + lane) * NBINS + g
                        plsc.store_scatter(out_t_v, [tgt], acc_lo[gc])
                        plsc.store_scatter(out_t_v, [tgt + HC * NBINS],
                                           acc_hi[gc])
                    return 0

                lax.fori_loop(0, BPC, bin_body, 0)
            pltpu.sync_copy(out_t_v, out_hbm.at[r])
            return 0

        lax.fori_loop(0, RPW, roi_body, 0)

    return roi_align_sc


def kernel(features, rois):
    N, C, H, W = features.shape
    R = rois.shape[0]
    lo16 = jax.lax.bitcast_convert_type(
        features[:, :C // 2].astype(jnp.bfloat16), jnp.uint16)
    hi16 = jax.lax.bitcast_convert_type(
        features[:, C // 2:].astype(jnp.bfloat16), jnp.uint16)
    word = lo16.astype(jnp.int32) | (hi16.astype(jnp.int32) << 16)
    feat = jnp.transpose(word, (0, 2, 3, 1)).reshape(N * H * W, C // 2)

    b = rois[:, 0].astype(jnp.int32)
    start_w = rois[:, 1] * SPATIAL_SCALE - 0.5
    start_h = rois[:, 2] * SPATIAL_SCALE - 0.5
    roi_w = rois[:, 3] * SPATIAL_SCALE - 0.5 - start_w
    roi_h = rois[:, 4] * SPATIAL_SCALE - 0.5 - start_h
    bin_h = roi_h / OUT_H
    bin_w = roi_w / OUT_W
    gh = gw = SAMPLING_RATIO
    ph = jnp.arange(OUT_H, dtype=jnp.float32)
    pw = jnp.arange(OUT_W, dtype=jnp.float32)
    iy = jnp.arange(gh, dtype=jnp.float32)
    ix = jnp.arange(gw, dtype=jnp.float32)
    y = (start_h[:, None, None] + ph[None, :, None] * bin_h[:, None, None]
         + (iy[None, None, :] + 0.5) * bin_h[:, None, None] / gh)
    x = (start_w[:, None, None] + pw[None, :, None] * bin_w[:, None, None]
         + (ix[None, None, :] + 0.5) * bin_w[:, None, None] / gw)
    P = OUT_H * OUT_W * gh * gw
    yy = jnp.broadcast_to(y[:, :, None, :, None],
                          (R, OUT_H, OUT_W, gh, gw)).reshape(R, P)
    xx = jnp.broadcast_to(x[:, None, :, None, :],
                          (R, OUT_H, OUT_W, gh, gw)).reshape(R, P)
    valid = (yy > -1.0) & (yy < H) & (xx > -1.0) & (xx < W)
    yc = jnp.clip(yy, 0.0, H - 1)
    xc = jnp.clip(xx, 0.0, W - 1)
    y_low = jnp.floor(yc).astype(jnp.int32)
    x_low = jnp.floor(xc).astype(jnp.int32)
    y_high = jnp.minimum(y_low + 1, H - 1)
    x_high = jnp.minimum(x_low + 1, W - 1)
    ly = yc - y_low.astype(jnp.float32)
    lx = xc - x_low.astype(jnp.float32)
    hy = 1.0 - ly
    hx = 1.0 - lx
    base = (b * (H * W))[:, None]
    idx4 = jnp.stack(
        [base + y_low * W + x_low, base + y_low * W + x_high,
         base + y_high * W + x_low, base + y_high * W + x_high], axis=-1)
    scale = jnp.where(valid, 1.0 / (gh * gw), 0.0)
    w4 = jnp.stack([hy * hx, hy * lx, ly * hx, ly * lx], axis=-1) * scale[:, :, None]

    NROWS = P * 4
    NCHUNK = 7
    idx_all = idx4.reshape(R, NCHUNK, NROWS // NCHUNK).astype(jnp.int32)
    w_all = w4.reshape(R, NCHUNK, NROWS // NCHUNK).astype(jnp.float32)

    sc_call = _make_sc_call(R, C, NROWS)
    out = sc_call(feat, idx_all, w_all)
    return out.reshape(R, C, OUT_H, OUT_W)
